# baseline (speedup 1.0000x reference)
import functools

import jax
import jax.numpy as jnp
from jax import lax
from jax.experimental import pallas as pl
from jax.experimental.pallas import tpu as pltpu
from jax.experimental.pallas import tpu_sc as plsc

N = 4096
WIDTH = 256
NHEAD = 8
KNN = 8
NB = 8
EMBED_POS = 16
WA = 512
GW = WIDTH // NHEAD
OW = WA // NHEAD

BLK = 512
QB = 512
TW = 512
TWI = TW // 2
E_PER_BLK = BLK * KNN
CW = 128
NC = N // CW


def _prep_kernel(x_ref, bf_ref, z_ref, ext_tab_ref, src_w_ref, keys_ref, tab_ref):
    x = x_ref[...]
    x0 = lax.dot_general(x, src_w_ref[...], (((1,), (1,)), ((), ())),
                         preferred_element_type=jnp.float32)
    bf = bf_ref[...]
    onehot = (bf == lax.broadcasted_iota(jnp.int32, (BLK, NB), 1
              ).astype(jnp.float32)).astype(jnp.float32)
    ext = jnp.dot(onehot, ext_tab_ref[...], preferred_element_type=jnp.float32)
    keys_ref[...] = jnp.concatenate([x0, ext], axis=1)
    pad = jnp.zeros((BLK, TW - WIDTH - EMBED_POS - 1), jnp.float32)
    tab_ref[...] = jnp.concatenate([x0, z_ref[...], bf, pad],
                                   axis=1).astype(jnp.bfloat16)


def _score_topk_kernel(x_ref, bf_ref, ext_tab_ref, tgt_w_ref, keys_ref,
                       x1_ref, idx_ref):
    x = x_ref[...]
    x1 = lax.dot_general(x, tgt_w_ref[...], (((1,), (1,)), ((), ())),
                         preferred_element_type=jnp.float32)
    x1_ref[...] = x1
    bf = bf_ref[...]
    onehot = (bf == lax.broadcasted_iota(jnp.int32, (QB, NB), 1
              ).astype(jnp.float32)).astype(jnp.float32)
    ext = jnp.dot(onehot, ext_tab_ref[...], preferred_element_type=jnp.float32)
    q = jnp.concatenate([x1, ext], axis=1)
    s = lax.dot_general(q, keys_ref[...], (((1,), (1,)), ((), ())),
                        preferred_element_type=jnp.float32)
    neg = jnp.float32(-jnp.inf)
    M = jnp.concatenate(
        [jnp.max(s[:, c * CW:(c + 1) * CW], axis=1, keepdims=True)
         for c in range(NC)], axis=1)
    ci = lax.broadcasted_iota(jnp.int32, (QB, NC), 1)
    li = lax.broadcasted_iota(jnp.int32, (QB, CW), 1)
    taken = []
    picks = []
    for _ in range(KNN):
        m1 = jnp.max(M, axis=1, keepdims=True)
        sel = jnp.min(jnp.where(M == m1, ci, NC), axis=1, keepdims=True)
        cd = jnp.full((QB, CW), neg)
        for c in range(NC):
            cd = jnp.maximum(cd, jnp.where(sel == c,
                                           s[:, c * CW:(c + 1) * CW], neg))
        for (tc_, tl_) in taken:
            cd = jnp.where((tc_ == sel) & (li == tl_), neg, cd)
        mv = jnp.max(cd, axis=1, keepdims=True)
        lane = jnp.min(jnp.where(cd == mv, li, CW), axis=1, keepdims=True)
        picks.append(sel * CW + lane)
        taken.append((sel, lane))
        newmax = jnp.max(jnp.where(li == lane, neg, cd), axis=1, keepdims=True)
        M = jnp.where(ci == sel, newmax, M)
    idx_ref[...] = jnp.concatenate(picks, axis=1)


def _gn_pool(v, pool_ref, expand_ref):
    sum_g = jnp.dot(v, pool_ref[...], preferred_element_type=jnp.float32)
    m = sum_g * (1.0 / GW)
    sq_g = jnp.dot(v * v, pool_ref[...], preferred_element_type=jnp.float32)
    var = sq_g * (1.0 / GW) - m * m
    m_e = jnp.dot(m, expand_ref[...], preferred_element_type=jnp.float32)
    v_e = jnp.dot(var, expand_ref[...], preferred_element_type=jnp.float32)
    return (v - m_e) * lax.rsqrt(v_e + 1e-5)


def _edge_kernel(base, g_ref, idx_ref, x1_ref, z_ref, bf_ref, xres_ref,
                 degp_ref, pre_t_ref, wg_pw_ref, wv_pw_ref, pwpost_t_ref,
                 wg_f_ref, wv_f_ref, fpost_t_ref, pool_ref, expand_ref,
                 out_ref):
    x1 = x1_ref[...]
    zq = z_ref[...]
    bf = bf_ref[...]
    ids = (jnp.int32(base) + jnp.int32(pl.program_id(0) * BLK)
           + lax.broadcasted_iota(jnp.int32, (BLK, 1), 0))
    bff = jnp.bfloat16
    wg_pw = wg_pw_ref[...].astype(bff)
    wv_pw = wv_pw_ref[...].astype(bff)
    wg_f = wg_f_ref[...].astype(bff)
    wv_f = wv_f_ref[...].astype(bff)
    pwpost = pwpost_t_ref[...].astype(bff)
    acc = jnp.zeros((BLK, WA), jnp.float32)
    deg = jnp.zeros((BLK, 1), jnp.float32)
    for k in range(KNN):
        slab = g_ref[k * BLK:(k + 1) * BLK, :]
        x0n = slab[:, :WIDTH].astype(jnp.float32)
        zn = slab[:, WIDTH:WIDTH + EMBED_POS].astype(jnp.float32)
        bn = slab[:, WIDTH + EMBED_POS:WIDTH + EMBED_POS + 1
                  ].astype(jnp.float32)
        idn = idx_ref[:, k:k + 1]
        keep = jnp.where((bn != bf) & (idn != ids), 1.0, 0.0)
        e_in = jnp.concatenate([zn, zq], axis=1)
        pre = jnp.dot(e_in, pre_t_ref[...], preferred_element_type=jnp.float32)
        xn = _gn_pool(pre, pool_ref, expand_ref).astype(bff)
        hb = (jax.nn.relu(jnp.dot(xn, wg_pw, preferred_element_type=jnp.float32))
              * jnp.dot(xn, wv_pw, preferred_element_type=jnp.float32)
              ).astype(bff)
        bias = jnp.dot(hb, pwpost, preferred_element_type=jnp.float32)
        mn = _gn_pool(x0n + x1, pool_ref, expand_ref)
        h = (jax.nn.relu(jnp.dot((mn + bias).astype(bff), wg_f,
                                 preferred_element_type=jnp.float32))
             * jnp.dot(mn.astype(bff), wv_f, preferred_element_type=jnp.float32))
        acc = acc + h * keep
        deg = deg + keep
    xx = jnp.dot(acc, fpost_t_ref[...], preferred_element_type=jnp.float32)
    deg = jnp.maximum(deg, 1.0)
    scale = jnp.exp(jnp.log(deg) * degp_ref[...])
    out_ref[...] = scale * xx + xres_ref[...]


def _sc_gather(tab, idx_flat):
    info = plsc.get_sparse_core_info()
    nw = info.num_cores * info.num_subcores
    ne = idx_flat.shape[0]
    b_per_w = ne // nw
    chunk = 128
    n_chunks = b_per_w // chunk
    mesh = plsc.VectorSubcoreMesh(core_axis_name="c", subcore_axis_name="s")

    @functools.partial(
        pl.kernel, mesh=mesh,
        out_type=jax.ShapeDtypeStruct((ne, TWI), jnp.int32),
        scratch_types=[
            pltpu.VMEM((b_per_w,), jnp.int32),
            pltpu.VMEM((chunk, TWI), jnp.int32),
            pltpu.VMEM((chunk, TWI), jnp.int32),
            pltpu.SemaphoreType.DMA,
        ],
    )
    def k(tab_hbm, idx_hbm, out_hbm, idx_v, buf0, buf1, sem):
        bufs = (buf0, buf1)
        wid = lax.axis_index("s") * info.num_cores + lax.axis_index("c")
        base = wid * b_per_w
        pltpu.sync_copy(idx_hbm.at[pl.ds(base, b_per_w)], idx_v)
        for r in range(0, n_chunks, 2):
            copies = [
                pltpu.async_copy(
                    tab_hbm.at[idx_v.at[pl.ds((r + j) * chunk, chunk)]],
                    bufs[j], sem)
                for j in range(2)
            ]
            for j in range(2):
                copies[j].wait()
            for j in range(2):
                pltpu.sync_copy(
                    bufs[j], out_hbm.at[pl.ds(base + (r + j) * chunk, chunk)])

    return k(tab, idx_flat)


def _block_diag(w):
    wr = w.reshape(NHEAD, OW, GW)
    eye = jnp.eye(NHEAD, dtype=w.dtype)
    m4 = eye[:, None, :, None] * wr.transpose(0, 2, 1)[:, :, None, :]
    return m4.reshape(WIDTH, WA)


def kernel(x, x_res, z_rw, z_3d, batch, batch_size, src_w, tgt_w, knn_ext,
           pw_pre, pw_gate, pw_val, pw_post, f_gate, f_val, f_post, deg_p):
    bf = batch.astype(jnp.float32).reshape(N, 1)

    keys, tab = pl.pallas_call(
        _prep_kernel,
        grid=(N // BLK,),
        in_specs=[
            pl.BlockSpec((BLK, WIDTH), lambda i: (i, 0)),
            pl.BlockSpec((BLK, 1), lambda i: (i, 0)),
            pl.BlockSpec((BLK, EMBED_POS), lambda i: (i, 0)),
            pl.BlockSpec((NB, EMBED_POS), lambda i: (0, 0)),
            pl.BlockSpec((WIDTH, WIDTH), lambda i: (0, 0)),
        ],
        out_specs=[
            pl.BlockSpec((BLK, WIDTH + EMBED_POS), lambda i: (i, 0)),
            pl.BlockSpec((BLK, TW), lambda i: (i, 0)),
        ],
        out_shape=[
            jax.ShapeDtypeStruct((N, WIDTH + EMBED_POS), jnp.float32),
            jax.ShapeDtypeStruct((N, TW), jnp.bfloat16),
        ],
        compiler_params=pltpu.CompilerParams(
            dimension_semantics=("parallel",)),
    )(x, bf, z_rw, knn_ext, src_w)

    tab_i = lax.bitcast_convert_type(tab.reshape(N, TWI, 2), jnp.int32)

    pre_t = pw_pre.T
    wg_pw = _block_diag(pw_gate)
    wv_pw = _block_diag(pw_val)
    wg_f = _block_diag(f_gate)
    wv_f = _block_diag(f_val)
    pwpost_t = pw_post.T
    fpost_t = f_post.T
    pool = (lax.broadcasted_iota(jnp.int32, (WIDTH, NHEAD), 0) // GW
            == lax.broadcasted_iota(jnp.int32, (WIDTH, NHEAD), 1)
            ).astype(jnp.float32)
    expand = pool.T
    degp = deg_p.reshape(1, WIDTH)

    half = N // 2
    outs = []
    for h in range(2):
        hq = h * (half // QB)
        hb = h * (half // BLK)
        x1, idx = pl.pallas_call(
            _score_topk_kernel,
            grid=(half // QB,),
            in_specs=[
                pl.BlockSpec((QB, WIDTH), lambda i, _o=hq: (i + _o, 0)),
                pl.BlockSpec((QB, 1), lambda i, _o=hq: (i + _o, 0)),
                pl.BlockSpec((NB, EMBED_POS), lambda i: (0, 0)),
                pl.BlockSpec((WIDTH, WIDTH), lambda i: (0, 0)),
                pl.BlockSpec((N, WIDTH + EMBED_POS), lambda i: (0, 0)),
            ],
            out_specs=[
                pl.BlockSpec((QB, WIDTH), lambda i: (i, 0)),
                pl.BlockSpec((QB, KNN), lambda i: (i, 0)),
            ],
            out_shape=[
                jax.ShapeDtypeStruct((half, WIDTH), jnp.float32),
                jax.ShapeDtypeStruct((half, KNN), jnp.int32),
            ],
            compiler_params=pltpu.CompilerParams(
                dimension_semantics=("parallel",)),
        )(x, bf, knn_ext, tgt_w, keys)

        idx_r = idx.reshape(half // BLK, BLK, KNN
                            ).transpose(0, 2, 1).reshape(-1)

        gi = _sc_gather(tab_i, idx_r)
        g = lax.bitcast_convert_type(gi, jnp.bfloat16).reshape(-1, TW)

        out_h = pl.pallas_call(
            functools.partial(_edge_kernel, h * half),
            grid=(half // BLK,),
            in_specs=[
                pl.BlockSpec((E_PER_BLK, TW), lambda i: (i, 0)),
                pl.BlockSpec((BLK, KNN), lambda i: (i, 0)),
                pl.BlockSpec((BLK, WIDTH), lambda i: (i, 0)),
                pl.BlockSpec((BLK, EMBED_POS), lambda i, _o=hb: (i + _o, 0)),
                pl.BlockSpec((BLK, 1), lambda i, _o=hb: (i + _o, 0)),
                pl.BlockSpec((BLK, WIDTH), lambda i, _o=hb: (i + _o, 0)),
                pl.BlockSpec((1, WIDTH), lambda i: (0, 0)),
                pl.BlockSpec((2 * EMBED_POS, WIDTH), lambda i: (0, 0)),
                pl.BlockSpec((WIDTH, WA), lambda i: (0, 0)),
                pl.BlockSpec((WIDTH, WA), lambda i: (0, 0)),
                pl.BlockSpec((WA, WIDTH), lambda i: (0, 0)),
                pl.BlockSpec((WIDTH, WA), lambda i: (0, 0)),
                pl.BlockSpec((WIDTH, WA), lambda i: (0, 0)),
                pl.BlockSpec((WA, WIDTH), lambda i: (0, 0)),
                pl.BlockSpec((WIDTH, NHEAD), lambda i: (0, 0)),
                pl.BlockSpec((NHEAD, WIDTH), lambda i: (0, 0)),
            ],
            out_specs=pl.BlockSpec((BLK, WIDTH), lambda i: (i, 0)),
            out_shape=jax.ShapeDtypeStruct((half, WIDTH), jnp.float32),
            compiler_params=pltpu.CompilerParams(
                dimension_semantics=("parallel",)),
        )(g, idx, x1, z_rw, bf, x_res, degp, pre_t, wg_pw, wv_pw, pwpost_t,
          wg_f, wv_f, fpost_t, pool, expand)
        outs.append(out_h)
    return jnp.concatenate(outs, axis=0)


# speedup vs baseline: 1.9914x; 1.9914x over previous
import functools

import jax
import jax.numpy as jnp
from jax import lax
from jax.experimental import pallas as pl
from jax.experimental.pallas import tpu as pltpu
from jax.experimental.pallas import tpu_sc as plsc

N = 4096
WIDTH = 256
NHEAD = 8
KNN = 8
NB = 8
EMBED_POS = 16
WA = 512
GW = WIDTH // NHEAD
OW = WA // NHEAD

BLK = 512
QB = 512
TW = 384
E_PER_BLK = BLK * KNN
CW = 128
NC = N // CW


def _prep_kernel(x_ref, bf_ref, z_ref, ext_tab_ref, src_w_ref, keys_ref, tab_ref):
    x = x_ref[...]
    x0 = lax.dot_general(x, src_w_ref[...], (((1,), (1,)), ((), ())),
                         preferred_element_type=jnp.float32)
    bf = bf_ref[...]
    onehot = (bf == lax.broadcasted_iota(jnp.int32, (BLK, NB), 1
              ).astype(jnp.float32)).astype(jnp.float32)
    ext = jnp.dot(onehot, ext_tab_ref[...], preferred_element_type=jnp.float32)
    keys_ref[...] = jnp.concatenate([x0, ext], axis=1)
    pad = jnp.zeros((BLK, TW - WIDTH - EMBED_POS - 1), jnp.float32)
    tab_ref[...] = jnp.concatenate([x0, z_ref[...], bf, pad], axis=1)


def _score_topk_kernel(x_ref, bf_ref, ext_tab_ref, tgt_w_ref, keys_ref,
                       x1_ref, idx_ref):
    x = x_ref[...]
    x1 = lax.dot_general(x, tgt_w_ref[...], (((1,), (1,)), ((), ())),
                         preferred_element_type=jnp.float32)
    x1_ref[...] = x1
    bf = bf_ref[...]
    onehot = (bf == lax.broadcasted_iota(jnp.int32, (QB, NB), 1
              ).astype(jnp.float32)).astype(jnp.float32)
    ext = jnp.dot(onehot, ext_tab_ref[...], preferred_element_type=jnp.float32)
    q = jnp.concatenate([x1, ext], axis=1)
    s = lax.dot_general(q, keys_ref[...], (((1,), (1,)), ((), ())),
                        preferred_element_type=jnp.float32)
    neg = jnp.float32(-jnp.inf)
    M = jnp.concatenate(
        [jnp.max(s[:, c * CW:(c + 1) * CW], axis=1, keepdims=True)
         for c in range(NC)], axis=1)
    ci = lax.broadcasted_iota(jnp.int32, (QB, NC), 1)
    li = lax.broadcasted_iota(jnp.int32, (QB, CW), 1)
    taken = []
    picks = []
    for _ in range(KNN):
        m1 = jnp.max(M, axis=1, keepdims=True)
        sel = jnp.min(jnp.where(M == m1, ci, NC), axis=1, keepdims=True)
        cd = jnp.full((QB, CW), neg)
        for c in range(NC):
            cd = jnp.maximum(cd, jnp.where(sel == c,
                                           s[:, c * CW:(c + 1) * CW], neg))
        for (tc_, tl_) in taken:
            cd = jnp.where((tc_ == sel) & (li == tl_), neg, cd)
        mv = jnp.max(cd, axis=1, keepdims=True)
        lane = jnp.min(jnp.where(cd == mv, li, CW), axis=1, keepdims=True)
        picks.append(sel * CW + lane)
        taken.append((sel, lane))
        newmax = jnp.max(jnp.where(li == lane, neg, cd), axis=1, keepdims=True)
        M = jnp.where(ci == sel, newmax, M)
    idx_ref[...] = jnp.concatenate(picks, axis=1)


def _gn_pool(v, pool_ref, expand_ref):
    sum_g = jnp.dot(v, pool_ref[...], preferred_element_type=jnp.float32)
    m = sum_g * (1.0 / GW)
    sq_g = jnp.dot(v * v, pool_ref[...], preferred_element_type=jnp.float32)
    var = sq_g * (1.0 / GW) - m * m
    rs = lax.rsqrt(var + 1e-5)
    m_e = jnp.dot(m, expand_ref[...], preferred_element_type=jnp.float32)
    rs_e = jnp.dot(rs, expand_ref[...], preferred_element_type=jnp.float32)
    return (v - m_e) * rs_e


def _edge_kernel(base, g_ref, idx_ref, x1_ref, z_ref, bf_ref, xres_ref,
                 degp_ref, pre_t_ref, wg_pw_ref, wv_pw_ref, pwpost_t_ref,
                 wg_f_ref, wv_f_ref, fpost_t_ref, pool_ref, expand_ref,
                 out_ref):
    x1 = x1_ref[...]
    zq = z_ref[...]
    bf = bf_ref[...]
    ids = (jnp.int32(base) + jnp.int32(pl.program_id(0) * BLK)
           + lax.broadcasted_iota(jnp.int32, (BLK, 1), 0))
    bff = jnp.bfloat16
    wg_pw = wg_pw_ref[...].astype(bff)
    wv_pw = wv_pw_ref[...].astype(bff)
    wg_f = wg_f_ref[...].astype(bff)
    wv_f = wv_f_ref[...].astype(bff)
    pwpost = pwpost_t_ref[...].astype(bff)
    acc = jnp.zeros((BLK, WA), jnp.float32)
    deg = jnp.zeros((BLK, 1), jnp.float32)
    for k in range(KNN):
        slab = g_ref[k * BLK:(k + 1) * BLK, :]
        x0n = slab[:, :WIDTH]
        zn = slab[:, WIDTH:WIDTH + EMBED_POS]
        bn = slab[:, WIDTH + EMBED_POS:WIDTH + EMBED_POS + 1]
        idn = idx_ref[:, k:k + 1]
        keep = jnp.where((bn != bf) & (idn != ids), 1.0, 0.0)
        e_in = jnp.concatenate([zn, zq], axis=1)
        pre = jnp.dot(e_in, pre_t_ref[...], preferred_element_type=jnp.float32)
        xn = _gn_pool(pre, pool_ref, expand_ref).astype(bff)
        hb = (jax.nn.relu(jnp.dot(xn, wg_pw, preferred_element_type=jnp.float32))
              * jnp.dot(xn, wv_pw, preferred_element_type=jnp.float32)
              ).astype(bff)
        bias = jnp.dot(hb, pwpost, preferred_element_type=jnp.float32)
        mn = _gn_pool(x0n + x1, pool_ref, expand_ref)
        h = (jax.nn.relu(jnp.dot((mn + bias).astype(bff), wg_f,
                                 preferred_element_type=jnp.float32))
             * jnp.dot(mn.astype(bff), wv_f, preferred_element_type=jnp.float32))
        acc = acc + h * keep
        deg = deg + keep
    xx = jnp.dot(acc, fpost_t_ref[...], preferred_element_type=jnp.float32)
    deg = jnp.maximum(deg, 1.0)
    scale = jnp.exp(jnp.log(deg) * degp_ref[...])
    out_ref[...] = scale * xx + xres_ref[...]


def _sc_gather(tab, idx_flat):
    info = plsc.get_sparse_core_info()
    nw = info.num_cores * info.num_subcores
    ne = idx_flat.shape[0]
    b_per_w = ne // nw
    chunk = 128
    n_chunks = b_per_w // chunk
    mesh = plsc.VectorSubcoreMesh(core_axis_name="c", subcore_axis_name="s")

    @functools.partial(
        pl.kernel, mesh=mesh,
        out_type=jax.ShapeDtypeStruct((ne, TW), jnp.float32),
        scratch_types=[
            pltpu.VMEM((b_per_w,), jnp.int32),
            pltpu.VMEM((chunk, TW), jnp.float32),
            pltpu.VMEM((chunk, TW), jnp.float32),
            pltpu.SemaphoreType.DMA,
        ],
    )
    def k(tab_hbm, idx_hbm, out_hbm, idx_v, buf0, buf1, sem):
        bufs = (buf0, buf1)
        wid = lax.axis_index("s") * info.num_cores + lax.axis_index("c")
        base = wid * b_per_w
        pltpu.sync_copy(idx_hbm.at[pl.ds(base, b_per_w)], idx_v)
        for r in range(0, n_chunks, 2):
            copies = [
                pltpu.async_copy(
                    tab_hbm.at[idx_v.at[pl.ds((r + j) * chunk, chunk)]],
                    bufs[j], sem)
                for j in range(2)
            ]
            for j in range(2):
                copies[j].wait()
            for j in range(2):
                pltpu.sync_copy(
                    bufs[j], out_hbm.at[pl.ds(base + (r + j) * chunk, chunk)])

    return k(tab, idx_flat)


def _block_diag(w):
    wr = w.reshape(NHEAD, OW, GW)
    eye = jnp.eye(NHEAD, dtype=w.dtype)
    m4 = eye[:, None, :, None] * wr.transpose(0, 2, 1)[:, :, None, :]
    return m4.reshape(WIDTH, WA)


def kernel(x, x_res, z_rw, z_3d, batch, batch_size, src_w, tgt_w, knn_ext,
           pw_pre, pw_gate, pw_val, pw_post, f_gate, f_val, f_post, deg_p):
    bf = batch.astype(jnp.float32).reshape(N, 1)

    keys, tab = pl.pallas_call(
        _prep_kernel,
        grid=(N // BLK,),
        in_specs=[
            pl.BlockSpec((BLK, WIDTH), lambda i: (i, 0)),
            pl.BlockSpec((BLK, 1), lambda i: (i, 0)),
            pl.BlockSpec((BLK, EMBED_POS), lambda i: (i, 0)),
            pl.BlockSpec((NB, EMBED_POS), lambda i: (0, 0)),
            pl.BlockSpec((WIDTH, WIDTH), lambda i: (0, 0)),
        ],
        out_specs=[
            pl.BlockSpec((BLK, WIDTH + EMBED_POS), lambda i: (i, 0)),
            pl.BlockSpec((BLK, TW), lambda i: (i, 0)),
        ],
        out_shape=[
            jax.ShapeDtypeStruct((N, WIDTH + EMBED_POS), jnp.float32),
            jax.ShapeDtypeStruct((N, TW), jnp.float32),
        ],
        compiler_params=pltpu.CompilerParams(
            dimension_semantics=("parallel",)),
    )(x, bf, z_rw, knn_ext, src_w)

    pre_t = pw_pre.T
    wg_pw = _block_diag(pw_gate)
    wv_pw = _block_diag(pw_val)
    wg_f = _block_diag(f_gate)
    wv_f = _block_diag(f_val)
    pwpost_t = pw_post.T
    fpost_t = f_post.T
    pool = (lax.broadcasted_iota(jnp.int32, (WIDTH, NHEAD), 0) // GW
            == lax.broadcasted_iota(jnp.int32, (WIDTH, NHEAD), 1)
            ).astype(jnp.float32)
    expand = pool.T
    degp = deg_p.reshape(1, WIDTH)

    half = N // 2
    outs = []
    for h in range(2):
        hq = h * (half // QB)
        hb = h * (half // BLK)
        x1, idx = pl.pallas_call(
            _score_topk_kernel,
            grid=(half // QB,),
            in_specs=[
                pl.BlockSpec((QB, WIDTH), lambda i, _o=hq: (i + _o, 0)),
                pl.BlockSpec((QB, 1), lambda i, _o=hq: (i + _o, 0)),
                pl.BlockSpec((NB, EMBED_POS), lambda i: (0, 0)),
                pl.BlockSpec((WIDTH, WIDTH), lambda i: (0, 0)),
                pl.BlockSpec((N, WIDTH + EMBED_POS), lambda i: (0, 0)),
            ],
            out_specs=[
                pl.BlockSpec((QB, WIDTH), lambda i: (i, 0)),
                pl.BlockSpec((QB, KNN), lambda i: (i, 0)),
            ],
            out_shape=[
                jax.ShapeDtypeStruct((half, WIDTH), jnp.float32),
                jax.ShapeDtypeStruct((half, KNN), jnp.int32),
            ],
            compiler_params=pltpu.CompilerParams(
                dimension_semantics=("parallel",)),
        )(x, bf, knn_ext, tgt_w, keys)

        idx_r = idx.reshape(half // BLK, BLK, KNN
                            ).transpose(0, 2, 1).reshape(-1)

        g = _sc_gather(tab, idx_r)

        out_h = pl.pallas_call(
            functools.partial(_edge_kernel, h * half),
            grid=(half // BLK,),
            in_specs=[
                pl.BlockSpec((E_PER_BLK, TW), lambda i: (i, 0)),
                pl.BlockSpec((BLK, KNN), lambda i: (i, 0)),
                pl.BlockSpec((BLK, WIDTH), lambda i: (i, 0)),
                pl.BlockSpec((BLK, EMBED_POS), lambda i, _o=hb: (i + _o, 0)),
                pl.BlockSpec((BLK, 1), lambda i, _o=hb: (i + _o, 0)),
                pl.BlockSpec((BLK, WIDTH), lambda i, _o=hb: (i + _o, 0)),
                pl.BlockSpec((1, WIDTH), lambda i: (0, 0)),
                pl.BlockSpec((2 * EMBED_POS, WIDTH), lambda i: (0, 0)),
                pl.BlockSpec((WIDTH, WA), lambda i: (0, 0)),
                pl.BlockSpec((WIDTH, WA), lambda i: (0, 0)),
                pl.BlockSpec((WA, WIDTH), lambda i: (0, 0)),
                pl.BlockSpec((WIDTH, WA), lambda i: (0, 0)),
                pl.BlockSpec((WIDTH, WA), lambda i: (0, 0)),
                pl.BlockSpec((WA, WIDTH), lambda i: (0, 0)),
                pl.BlockSpec((WIDTH, NHEAD), lambda i: (0, 0)),
                pl.BlockSpec((NHEAD, WIDTH), lambda i: (0, 0)),
            ],
            out_specs=pl.BlockSpec((BLK, WIDTH), lambda i: (i, 0)),
            out_shape=jax.ShapeDtypeStruct((half, WIDTH), jnp.float32),
            compiler_params=pltpu.CompilerParams(
                dimension_semantics=("parallel",)),
        )(g, idx, x1, z_rw, bf, x_res, degp, pre_t, wg_pw, wv_pw, pwpost_t,
          wg_f, wv_f, fpost_t, pool, expand)
        outs.append(out_h)
    return jnp.concatenate(outs, axis=0)
